# baseline (device time: 8641 ns/iter reference)
import jax
import jax.numpy as jnp
from jax import lax
from jax.experimental import pallas as pl
from jax.experimental.pallas import tpu as pltpu

NX, NY, NZ = 2, 4, 4
S = 48

PEER_SLOT = [1, 0, 3, 2, 5, 4]

CDT = jnp.bfloat16


def kernel(u):
    def body(u_ref, o_ref, sbuf, rbuf, send_sems, recv_sems):
        ix = lax.axis_index("x")
        iy = lax.axis_index("y")
        iz = lax.axis_index("z")

        conds = [ix > 0, ix < NX - 1, iy > 0, iy < NY - 1, iz > 0, iz < NZ - 1]
        targets = [
            (ix - 1, iy, iz),
            (ix + 1, iy, iz),
            (ix, iy - 1, iz),
            (ix, iy + 1, iz),
            (ix, iy, iz - 1),
            (ix, iy, iz + 1),
        ]

        barrier_sem = pltpu.get_barrier_semaphore()
        for s in range(6):
            @pl.when(conds[s])
            def _(s=s):
                pl.semaphore_signal(
                    barrier_sem, inc=1,
                    device_id=targets[s],
                    device_id_type=pl.DeviceIdType.MESH,
                )

        uval = u_ref[...].astype(CDT)

        sbuf[0, :, :] = uval[0, :, :]
        sbuf[1, :, :] = uval[S - 1, :, :]
        sbuf[2, :, :] = uval[:, 0, :]
        sbuf[3, :, :] = uval[:, S - 1, :]
        sbuf[4, :, :] = uval[:, :, 0]
        sbuf[5, :, :] = uval[:, :, S - 1]

        n_nbr = sum(c.astype(jnp.int32) for c in conds)
        pl.semaphore_wait(barrier_sem, n_nbr)

        def descriptor(slot, target):
            return pltpu.make_async_remote_copy(
                src_ref=sbuf.at[slot],
                dst_ref=rbuf.at[PEER_SLOT[slot]],
                send_sem=send_sems.at[slot],
                recv_sem=recv_sems.at[PEER_SLOT[slot]],
                device_id=target,
                device_id_type=pl.DeviceIdType.MESH,
            )

        for s in range(6):
            @pl.when(conds[s])
            def _(s=s):
                descriptor(s, targets[s]).start()

        zero2 = jnp.zeros((S, S), CDT)
        for r in range(6):
            @pl.when(jnp.logical_not(conds[r]))
            def _(r=r):
                rbuf[r, :, :] = zero2

        for r in range(6):
            @pl.when(conds[r])
            def _(r=r):
                pltpu.make_async_remote_copy(
                    src_ref=sbuf.at[0],
                    dst_ref=rbuf.at[r],
                    send_sem=send_sems.at[0],
                    recv_sem=recv_sems.at[r],
                    device_id=(ix, iy, iz),
                    device_id_type=pl.DeviceIdType.MESH,
                ).wait_recv()

        v = (
            jnp.concatenate([rbuf[0, :, :][None], uval[:-1, :, :]], axis=0)
            + jnp.concatenate([uval[1:, :, :], rbuf[1, :, :][None]], axis=0)
            + jnp.concatenate([rbuf[2, :, :][:, None, :], uval[:, :-1, :]], axis=1)
            + jnp.concatenate([uval[:, 1:, :], rbuf[3, :, :][:, None, :]], axis=1)
            + jnp.concatenate([rbuf[4, :, :][:, :, None], uval[:, :, :-1]], axis=2)
            + jnp.concatenate([uval[:, :, 1:], rbuf[5, :, :][:, :, None]], axis=2)
            - CDT(6.0) * uval
        )
        o_ref[...] = v.astype(o_ref.dtype)

        zplane = jnp.zeros((S, S), o_ref.dtype)

        @pl.when(ix == 0)
        def _():
            o_ref[0, :, :] = zplane

        @pl.when(ix == NX - 1)
        def _():
            o_ref[S - 1, :, :] = zplane

        @pl.when(iy == 0)
        def _():
            o_ref[:, 0, :] = zplane

        @pl.when(iy == NY - 1)
        def _():
            o_ref[:, S - 1, :] = zplane

        @pl.when(iz == 0)
        def _():
            o_ref[:, :, 0] = zplane

        @pl.when(iz == NZ - 1)
        def _():
            o_ref[:, :, S - 1] = zplane

        for s in range(6):
            @pl.when(conds[s])
            def _(s=s):
                descriptor(s, targets[s]).wait_send()

    return pl.pallas_call(
        body,
        out_shape=jax.ShapeDtypeStruct((S, S, S), u.dtype),
        in_specs=[pl.BlockSpec(memory_space=pltpu.VMEM)],
        out_specs=pl.BlockSpec(memory_space=pltpu.VMEM),
        scratch_shapes=[
            pltpu.VMEM((6, S, S), CDT),
            pltpu.VMEM((6, S, S), CDT),
            pltpu.SemaphoreType.DMA((6,)),
            pltpu.SemaphoreType.DMA((6,)),
        ],
        compiler_params=pltpu.CompilerParams(collective_id=0),
    )(u)
